# baseline (speedup 1.0000x reference)
import jax
import jax.numpy as jnp
from jax.experimental.compute_on import compute_on
from jax.experimental import pallas as pl


@compute_on("tpu_sparsecore")
@jax.jit
def _take(weight, idx):
    return jnp.take(weight, idx, axis=0, mode="clip")


def kernel(input, weight):
    n = input.size
    out = _take(weight, input.reshape(n).astype(jnp.int32))
    return out.reshape(input.shape + (weight.shape[1],))


# speedup vs baseline: 1.9844x; 1.9844x over previous
import functools

import jax
import jax.numpy as jnp
from jax import lax
from jax.experimental import pallas as pl
from jax.experimental.pallas import tpu as pltpu
from jax.experimental.pallas import tpu_sc as plsc

_info = plsc.get_sparse_core_info()
_NC = _info.num_cores
_NS = _info.num_subcores
_NW = _NC * _NS

_CHUNK = 128
_NBUF = 8


@functools.partial(jax.jit, static_argnames=("n_rows", "dim"))
def _sc_gather(weight, idx, *, n_rows, dim):
    b_per_w = n_rows // _NW
    n_chunks = b_per_w // _CHUNK
    n_groups = n_chunks // _NBUF
    mesh = plsc.VectorSubcoreMesh(core_axis_name="c", subcore_axis_name="s")

    @functools.partial(
        pl.kernel,
        mesh=mesh,
        out_type=jax.ShapeDtypeStruct((n_rows, dim), jnp.float32),
        scratch_types=[
            pltpu.VMEM((b_per_w,), jnp.int32),
            pltpu.VMEM((_NBUF, _CHUNK, dim), jnp.float32),
            pltpu.SemaphoreType.DMA((_NBUF,)),
            pltpu.SemaphoreType.DMA((_NBUF,)),
        ],
        compiler_params=pltpu.CompilerParams(use_tc_tiling_on_sc=False),
    )
    def k(table_hbm, idx_hbm, out_hbm, idx_v, rows_v, gsem, ssem):
        wid = lax.axis_index("s") * _NC + lax.axis_index("c")
        base = wid * b_per_w
        pltpu.sync_copy(idx_hbm.at[pl.ds(base, b_per_w)], idx_v)

        def gather_copy(chunk, b):
            off = pl.multiple_of(chunk * _CHUNK, 8)
            return pltpu.make_async_copy(
                table_hbm.at[idx_v.at[pl.ds(off, _CHUNK)]],
                rows_v.at[b],
                gsem.at[b],
            )

        def store_copy(chunk, b):
            off = pl.multiple_of(chunk * _CHUNK, 8)
            return pltpu.make_async_copy(
                rows_v.at[b],
                out_hbm.at[pl.ds(base + off, _CHUNK), :],
                ssem.at[b],
            )

        for b in range(_NBUF):
            gather_copy(b, b).start()

        def group(grp, carry):
            for b in range(_NBUF):
                chunk = grp * _NBUF + b
                gather_copy(chunk, b).wait()
                store_copy(chunk, b).start()
                store_copy(chunk, b).wait()

                @pl.when(grp < n_groups - 1)
                def _():
                    gather_copy(chunk + _NBUF, b).start()

            return carry

        lax.fori_loop(0, n_groups, group, 0)

    return k(weight, idx)


def kernel(input, weight):
    n_rows = input.size
    dim = weight.shape[1]
    idx = input.reshape(n_rows).astype(jnp.int32)
    out = _sc_gather(weight, idx, n_rows=n_rows, dim=dim)
    return out.reshape(input.shape + (dim,))
